# baseline (speedup 1.0000x reference)
import jax
import jax.numpy as jnp
from jax.experimental import pallas as pl

_HB = 32
_TT = (
    (0, 2, 120, 128),
    (1, 2, 0, 12),
    (2, 1, 94, 114),
    (4, 0, 48, 68),
)


def _tc_body(yref, oref):
    tt = pl.program_id(2)
    lanes = jax.lax.broadcasted_iota(jnp.int32, (1, 1, 128), 2)
    x = yref[0]

    total = jnp.zeros((_HB, 224), jnp.float32)
    for i, (_, _, lo, hi) in enumerate(_TT):
        m = (lanes >= lo) & (lanes < hi)
        s = jnp.sum(jnp.where(m, x, 0.0), axis=-1)
        total = jnp.where(tt == i, s, total)
    total = total * (1.0 / 20.0)

    @pl.when(tt != 1)
    def _():
        oref[0, 0] = total

    @pl.when(tt == 1)
    def _():
        oref[0, 0] += total


def _ctile(tt):
    return jnp.where(tt == 0, 0,
                     jnp.where(tt == 1, 1, jnp.where(tt == 2, 2, 4)))


def _win(tt):
    return jnp.where(tt < 2, 2, jnp.where(tt == 2, 1, 0))


def kernel(x):
    B, C, H, W = x.shape
    y = jnp.transpose(x, (0, 2, 3, 1))

    out = pl.pallas_call(
        _tc_body,
        grid=(B, H // _HB, len(_TT)),
        in_specs=[
            pl.BlockSpec(
                (1, _HB, W, 128),
                lambda b, h, tt: (b, h, 0, _ctile(tt)),
            )
        ],
        out_specs=pl.BlockSpec(
            (1, 1, _HB, W),
            lambda b, h, tt: (b, _win(tt), h, 0),
        ),
        out_shape=jax.ShapeDtypeStruct((B, 3, H, W), jnp.float32),
    )(y)
    return out


# speedup vs baseline: 2.0102x; 2.0102x over previous
import functools

import jax
import jax.numpy as jnp
from jax import lax
from jax.experimental import pallas as pl
from jax.experimental.pallas import tpu as pltpu
from jax.experimental.pallas import tpu_sc as plsc

_WIN_BASES = (560, 350, 120)
_NWIN = 3
_WLEN = 20
_NC = 2
_NS = 16
_NW = _NC * _NS
_LANES = 16
_WCH = 112
_HOCT = 8
_HSC = 128
_HB = 32

_SLABS = ((0, 256, 0), (256, 128, 256), (512, 128, 384))
_DST_BASE = (432, 350, 120)

_TT = (
    (0, 2, 120, 128),
    (1, 2, 0, 12),
    (2, 1, 94, 114),
    (4, 0, 48, 68),
)


def _sc_body(y_hbm, out_hbm, buf_v, res_v, sem0, sem1, osem):
    B, H, W, C = y_hbm.shape
    noct = _HSC // _HOCT
    nwc = W // _WCH
    nsub = _HOCT * nwc
    ng = _WCH // _LANES
    inv = jnp.float32(1.0 / _WLEN)
    sems = (sem0, sem1)

    c = lax.axis_index("c")
    s = lax.axis_index("s")
    wid = s * _NC + c

    t = wid
    b = t // noct
    h0 = pl.multiple_of((t % noct) * _HOCT, _HOCT)

    def coords(m):
        i = m % nsub
        hof = i // nwc
        wc = i % nwc
        return hof, wc

    def copies(m):
        hof, wc = coords(m)
        w0 = pl.multiple_of(wc * _WCH, _WCH)
        for (so, width, do) in _SLABS:
            yield (
                y_hbm.at[b, h0 + hof, pl.ds(w0, _WCH), pl.ds(so, width)],
                buf_v.at[m % 2, :, pl.ds(do, width)],
                sems,
                m % 2,
            )

    def issue(m):
        for (src, dst, ss, par) in copies(m):

            @pl.when(par == 0)
            def _():
                pltpu.async_copy(src, dst, ss[0])

            @pl.when(par == 1)
            def _():
                pltpu.async_copy(src, dst, ss[1])

    def drain_in(m):
        for (src, dst, ss, par) in copies(m):

            @pl.when(par == 0)
            def _():
                pltpu.make_async_copy(src, dst, ss[0]).wait()

            @pl.when(par == 1)
            def _():
                pltpu.make_async_copy(src, dst, ss[1]).wait()

    iota = lax.iota(jnp.int32, _LANES)

    issue(0)

    def step(m, carry):
        hof, wc = coords(m)

        @pl.when(m + 1 < nsub)
        def _():
            issue(m + 1)

        drain_in(m)

        def group(g, carry2):
            idx_w = g * _LANES + iota
            for wi in range(_NWIN):
                acc = plsc.load_gather(
                    buf_v.at[m % 2],
                    [idx_w, jnp.full((_LANES,), _DST_BASE[wi], jnp.int32)],
                )
                for j in range(1, _WLEN):
                    acc = acc + plsc.load_gather(
                        buf_v.at[m % 2],
                        [idx_w,
                         jnp.full((_LANES,), _DST_BASE[wi] + j, jnp.int32)],
                    )
                res_v[0, wi, hof,
                      pl.ds(wc * _WCH + g * _LANES, _LANES)] = acc * inv
            return carry2

        lax.fori_loop(0, ng, group, 0)
        return carry

    lax.fori_loop(0, nsub, step, 0)

    for wi in range(_NWIN):
        pltpu.async_copy(
            res_v.at[0, wi], out_hbm.at[b, wi, pl.ds(h0, _HOCT), :], osem
        )
    for wi in range(_NWIN):
        pltpu.make_async_copy(
            res_v.at[0, wi], out_hbm.at[b, wi, pl.ds(h0, _HOCT), :], osem
        ).wait()


def _tc_body(yref, oref):
    tt = pl.program_id(2)
    hb, w = yref.shape[1], yref.shape[2]

    c_idx = lax.broadcasted_iota(jnp.int32, (128, 8), 0)
    v_idx = lax.broadcasted_iota(jnp.int32, (128, 8), 1)
    win = jnp.where(tt < 2, 2, jnp.where(tt == 2, 1, 0))
    lo = jnp.where(tt == 0, 120, jnp.where(tt == 1, 0,
                   jnp.where(tt == 2, 94, 48)))
    hi = jnp.where(tt == 0, 128, jnp.where(tt == 1, 12,
                   jnp.where(tt == 2, 114, 68)))
    m = (v_idx == win) & (c_idx >= lo) & (c_idx < hi)
    sel = jnp.where(m, jnp.float32(1.0 / _WLEN), jnp.float32(0.0))

    x2 = yref[0].reshape(hb * w, 128)
    s = jax.lax.dot_general(
        x2, sel, (((1,), (0,)), ((), ())),
        preferred_element_type=jnp.float32,
    ).reshape(hb, w, 8)

    @pl.when(tt == 0)
    def _():
        oref[0] = s

    @pl.when(tt > 0)
    def _():
        oref[0] += s


def _ctile(tt):
    return jnp.where(tt == 0, 0,
                     jnp.where(tt == 1, 1, jnp.where(tt == 2, 2, 4)))


def kernel(x):
    B, C, H, W = x.shape
    y = jnp.transpose(x, (0, 2, 3, 1))
    htc = H - _HSC

    sc_run = functools.partial(
        pl.kernel,
        out_type=jax.ShapeDtypeStruct((B, _NWIN, _HSC, W), jnp.float32),
        mesh=plsc.VectorSubcoreMesh(core_axis_name="c", subcore_axis_name="s"),
        compiler_params=pltpu.CompilerParams(needs_layout_passes=False),
        scratch_types=[
            pltpu.VMEM((2, _WCH, 512), jnp.float32),
            pltpu.VMEM((1, _NWIN, _HOCT, W), jnp.float32),
            pltpu.SemaphoreType.DMA,
            pltpu.SemaphoreType.DMA,
            pltpu.SemaphoreType.DMA,
        ],
    )(_sc_body)

    sc_out = sc_run(y)

    hsc_blocks = _HSC // _HB
    tc_out4 = pl.pallas_call(
        _tc_body,
        grid=(B, htc // _HB, len(_TT)),
        in_specs=[
            pl.BlockSpec(
                (1, _HB, W, 128),
                lambda b, h, tt: (b, hsc_blocks + h, 0, _ctile(tt)),
            )
        ],
        out_specs=pl.BlockSpec(
            (1, _HB, W, 8),
            lambda b, h, tt: (b, h, 0, 0),
        ),
        out_shape=jax.ShapeDtypeStruct((B, htc, W, 8), jnp.float32),
    )(y)
    tc_out = jnp.transpose(tc_out4[:, :, :, :_NWIN], (0, 3, 1, 2))

    return jnp.concatenate([sc_out, tc_out], axis=2)
